# Initial kernel scaffold; baseline (speedup 1.0000x reference)
import jax
import jax.numpy as jnp
from jax.experimental import pallas as pl


def kernel(inputs, query, key, value, outputs, gate, key_memories, value_memories):
    raise NotImplementedError("write your pallas kernel here")


# speedup vs baseline: 135.3935x; 135.3935x over previous
import functools

import jax
import jax.numpy as jnp
from jax.experimental import pallas as pl
from jax.experimental.pallas import tpu as pltpu

_EPS = 1e-8
_K = 8
_NEG = float("-inf")


def _halves(x):
    w = x.shape[1]
    return x[:, : w // 2], x[:, w // 2:]


def _fold_single(s):
    a, b = _halves(s)
    return [jnp.maximum(a, b), jnp.minimum(a, b)]


def _fold_2list(lst):
    h, l = lst
    ha, hb = _halves(h)
    la, lb = _halves(l)
    hi1 = jnp.maximum(ha, hb)
    lo1 = jnp.minimum(ha, hb)
    hi2 = jnp.maximum(la, lb)
    lo2 = jnp.minimum(la, lb)
    return [hi1, jnp.maximum(lo1, hi2), jnp.minimum(lo1, hi2), lo2]


def _merge_4lists(a, b):
    d1 = jnp.maximum(a[0], b[3])
    d2 = jnp.maximum(a[1], b[2])
    d3 = jnp.maximum(a[2], b[1])
    d4 = jnp.maximum(a[3], b[0])
    e1 = jnp.maximum(d1, d3)
    e3 = jnp.minimum(d1, d3)
    e2 = jnp.maximum(d2, d4)
    e4 = jnp.minimum(d2, d4)
    return [jnp.maximum(e1, e2), jnp.minimum(e1, e2),
            jnp.maximum(e3, e4), jnp.minimum(e3, e4)]


def _fold_4list(lst):
    ah, bh = zip(*(_halves(x) for x in lst))
    return _merge_4lists(list(ah), list(bh))


def _tile_top4(s, nclass):
    lst = _fold_single(s)
    lst = _fold_2list(lst)
    while lst[0].shape[1] > nclass:
        lst = _fold_4list(lst)
    return lst


def _body(gate_ref, q_ref, km_ref, vm_ref, outp_ref, o_ref, sims_ref, kn_ref,
          vbf_ref, stage_ref, dma_sem, *, H, S, Q, M, d, QBLK, MTILE):
    h = pl.program_id(0)
    qb = pl.program_id(1)
    NT = M // MTILE

    @pl.when(qb == 0)
    def _prep():
        for t in range(NT):
            sl = pl.ds(t * MTILE, MTILE)
            cp = pltpu.make_async_copy(km_ref.at[h, sl, :], stage_ref, dma_sem)
            cp.start()
            cp.wait()
            km = stage_ref[...]
            kn_ref[sl, :] = km / jnp.maximum(
                jnp.sqrt(jnp.sum(km * km, axis=1, keepdims=True)), _EPS)
            cp = pltpu.make_async_copy(vm_ref.at[h, sl, :], stage_ref, dma_sem)
            cp.start()
            cp.wait()
            vbf_ref[sl, :] = stage_ref[...].astype(jnp.bfloat16)

    q = q_ref[0, 0]
    qn = q / jnp.maximum(jnp.sqrt(jnp.sum(q * q, axis=1, keepdims=True)), _EPS)

    top4 = None
    for t in range(NT):
        kn = kn_ref[t * MTILE:(t + 1) * MTILE, :]
        s = jax.lax.dot_general(qn, kn, (((1,), (1,)), ((), ())),
                                preferred_element_type=jnp.float32)
        sims_ref[:, t * MTILE:(t + 1) * MTILE] = s
        t4 = _tile_top4(s, 128)
        top4 = t4 if top4 is None else _merge_4lists(top4, t4)

    cand = jnp.concatenate(top4, axis=1)
    thr = jnp.full((QBLK, 1), jnp.inf, dtype=jnp.float32)
    for _ in range(_K):
        thr = jnp.max(jnp.where(cand < thr, cand, _NEG), axis=1, keepdims=True)

    acc = jnp.zeros((QBLK, d), jnp.float32)
    for t in range(NT):
        s = sims_ref[:, t * MTILE:(t + 1) * MTILE]
        sel = jnp.where(s >= thr, s, 0.0).astype(jnp.bfloat16)
        v = vbf_ref[t * MTILE:(t + 1) * MTILE, :]
        acc = acc + jax.lax.dot_general(sel, v, (((1,), (0,)), ((), ())),
                                        preferred_element_type=jnp.float32)

    i0 = h * Q + qb * QBLK
    h2 = (i0 % (H * S)) // S
    g = jax.nn.sigmoid(gate_ref[h2])
    o_ref[0, 0] = g * acc + (1.0 - g) * outp_ref[0, 0]


def kernel(inputs, query, key, value, outputs, gate, key_memories, value_memories):
    del inputs, key, value
    B, H, S, d = query.shape
    _, M, _ = key_memories.shape
    Q = B * S
    QBLK = min(256, S)
    MTILE = min(4096, M)
    HS = H * S

    def q_map(h, qb):
        qi = qb * QBLK
        return (qi // S, h, (qi % S) // QBLK, 0)

    def o_map(h, qb):
        i0 = h * Q + qb * QBLK
        return (i0 // HS, (i0 % HS) // S, (i0 % S) // QBLK, 0)

    grid = (H, Q // QBLK)
    out = pl.pallas_call(
        functools.partial(_body, H=H, S=S, Q=Q, M=M, d=d, QBLK=QBLK, MTILE=MTILE),
        grid=grid,
        in_specs=[
            pl.BlockSpec(memory_space=pltpu.SMEM),
            pl.BlockSpec((1, 1, QBLK, d), q_map),
            pl.BlockSpec(memory_space=pltpu.MemorySpace.HBM),
            pl.BlockSpec(memory_space=pltpu.MemorySpace.HBM),
            pl.BlockSpec((1, 1, QBLK, d), o_map),
        ],
        out_specs=pl.BlockSpec((1, 1, QBLK, d), o_map),
        out_shape=jax.ShapeDtypeStruct((B, H, S, d), jnp.float32),
        scratch_shapes=[pltpu.VMEM((QBLK, M), jnp.float32),
                        pltpu.VMEM((M, d), jnp.float32),
                        pltpu.VMEM((M, d), jnp.bfloat16),
                        pltpu.VMEM((MTILE, d), jnp.float32),
                        pltpu.SemaphoreType.DMA],
        compiler_params=pltpu.CompilerParams(
            dimension_semantics=("arbitrary", "arbitrary")),
    )(gate, query, key_memories, value_memories, outputs)
    return out


# speedup vs baseline: 137.3790x; 1.0147x over previous
import functools

import jax
import jax.numpy as jnp
from jax.experimental import pallas as pl
from jax.experimental.pallas import tpu as pltpu

_EPS = 1e-8
_K = 8
_NEG = float("-inf")


def _halves(x):
    w = x.shape[1]
    return x[:, : w // 2], x[:, w // 2:]


def _fold_single(s):
    a, b = _halves(s)
    return [jnp.maximum(a, b), jnp.minimum(a, b)]


def _fold_2list(lst):
    h, l = lst
    ha, hb = _halves(h)
    la, lb = _halves(l)
    hi1 = jnp.maximum(ha, hb)
    lo1 = jnp.minimum(ha, hb)
    hi2 = jnp.maximum(la, lb)
    lo2 = jnp.minimum(la, lb)
    return [hi1, jnp.maximum(lo1, hi2), jnp.minimum(lo1, hi2), lo2]


def _merge_4lists(a, b):
    d1 = jnp.maximum(a[0], b[3])
    d2 = jnp.maximum(a[1], b[2])
    d3 = jnp.maximum(a[2], b[1])
    d4 = jnp.maximum(a[3], b[0])
    e1 = jnp.maximum(d1, d3)
    e3 = jnp.minimum(d1, d3)
    e2 = jnp.maximum(d2, d4)
    e4 = jnp.minimum(d2, d4)
    return [jnp.maximum(e1, e2), jnp.minimum(e1, e2),
            jnp.maximum(e3, e4), jnp.minimum(e3, e4)]


def _fold_4list(lst):
    ah, bh = zip(*(_halves(x) for x in lst))
    return _merge_4lists(list(ah), list(bh))


def _tile_top4(s, nclass):
    lst = _fold_single(s)
    lst = _fold_2list(lst)
    while lst[0].shape[1] > nclass:
        lst = _fold_4list(lst)
    return lst


def _body(gate_ref, q_ref, km_ref, vm_ref, outp_ref, o_ref, sims_ref, thr_ref,
          kn_ref, vbf_ref, stage_ref, dma_sem, *,
          H, S, Q, M, d, QBLK, MTILE, NSTEP):
    pid = pl.program_id(0)
    NT = M // MTILE
    nqb = Q // QBLK

    j = jnp.maximum(pid - 1, 0)
    parc = j % 2
    thr_c = thr_ref[parc]
    acc = jnp.zeros((QBLK, d), jnp.float32)
    for t in range(NT):
        s = sims_ref[parc, :, t * MTILE:(t + 1) * MTILE]
        sel = jnp.where(s >= thr_c, s, 0.0).astype(jnp.bfloat16)
        v = vbf_ref[t * MTILE:(t + 1) * MTILE, :]
        acc = acc + jax.lax.dot_general(
            sel, v, (((1,), (0,)), ((), ())),
            preferred_element_type=jnp.float32)
    i0 = j * QBLK
    h2 = (i0 % (H * S)) // S
    g = jax.nn.sigmoid(gate_ref[h2])
    o_ref[0, 0] = g * acc + (1.0 - g) * outp_ref[0, 0]

    h = jnp.minimum(pid, NSTEP - 2) // nqb
    qb = jnp.minimum(pid, NSTEP - 2) % nqb

    @pl.when(qb == 0)
    def _prep():
        for t in range(NT):
            sl = pl.ds(t * MTILE, MTILE)
            cp = pltpu.make_async_copy(km_ref.at[h, sl, :], stage_ref, dma_sem)
            cp.start()
            cp.wait()
            km = stage_ref[...]
            kn_ref[sl, :] = km / jnp.maximum(
                jnp.sqrt(jnp.sum(km * km, axis=1, keepdims=True)), _EPS)
            cp = pltpu.make_async_copy(vm_ref.at[h, sl, :], stage_ref, dma_sem)
            cp.start()
            cp.wait()
            vbf_ref[sl, :] = stage_ref[...].astype(jnp.bfloat16)

    par = jnp.minimum(pid, NSTEP - 2) % 2
    q = q_ref[0, 0]
    qn = q / jnp.maximum(jnp.sqrt(jnp.sum(q * q, axis=1, keepdims=True)), _EPS)
    top4 = None
    for t in range(NT):
        kn = kn_ref[t * MTILE:(t + 1) * MTILE, :]
        s = jax.lax.dot_general(qn, kn, (((1,), (1,)), ((), ())),
                                preferred_element_type=jnp.float32)
        sims_ref[par, :, t * MTILE:(t + 1) * MTILE] = s
        t4 = _tile_top4(s, 128)
        top4 = t4 if top4 is None else _merge_4lists(top4, t4)

    cand = jnp.concatenate(top4, axis=1)
    thr = jnp.full((QBLK, 1), jnp.inf, dtype=jnp.float32)
    for _ in range(_K):
        thr = jnp.max(jnp.where(cand < thr, cand, _NEG), axis=1,
                      keepdims=True)
    thr_ref[par] = thr


def kernel(inputs, query, key, value, outputs, gate, key_memories, value_memories):
    del inputs, key, value
    B, H, S, d = query.shape
    _, M, _ = key_memories.shape
    Q = B * S
    QBLK = min(256, S)
    MTILE = min(4096, M)
    HS = H * S
    nqb = Q // QBLK
    NSTEP = H * nqb + 1

    def q_map(pid):
        i = jnp.minimum(pid, H * nqb - 1)
        h, qb = i // nqb, i % nqb
        qi = qb * QBLK
        return (qi // S, h, (qi % S) // QBLK, 0)

    def o_map(pid):
        j = jnp.maximum(pid - 1, 0)
        i0 = j * QBLK
        return (i0 // HS, (i0 % HS) // S, (i0 % S) // QBLK, 0)

    out = pl.pallas_call(
        functools.partial(_body, H=H, S=S, Q=Q, M=M, d=d, QBLK=QBLK,
                          MTILE=MTILE, NSTEP=NSTEP),
        grid=(NSTEP,),
        in_specs=[
            pl.BlockSpec(memory_space=pltpu.SMEM),
            pl.BlockSpec((1, 1, QBLK, d), q_map),
            pl.BlockSpec(memory_space=pltpu.MemorySpace.HBM),
            pl.BlockSpec(memory_space=pltpu.MemorySpace.HBM),
            pl.BlockSpec((1, 1, QBLK, d), o_map),
        ],
        out_specs=pl.BlockSpec((1, 1, QBLK, d), o_map),
        out_shape=jax.ShapeDtypeStruct((B, H, S, d), jnp.float32),
        scratch_shapes=[pltpu.VMEM((2, QBLK, M), jnp.float32),
                        pltpu.VMEM((2, QBLK, 1), jnp.float32),
                        pltpu.VMEM((M, d), jnp.float32),
                        pltpu.VMEM((M, d), jnp.bfloat16),
                        pltpu.VMEM((MTILE, d), jnp.float32),
                        pltpu.SemaphoreType.DMA],
        compiler_params=pltpu.CompilerParams(
            dimension_semantics=("arbitrary",)),
    )(gate, query, key_memories, value_memories, outputs)
    return out
